# Initial kernel scaffold; baseline (speedup 1.0000x reference)
import jax
import jax.numpy as jnp
from jax.experimental import pallas as pl


def kernel(x, edge_index, W0, b0, g0, be0, W1, b1, g1, be1):
    raise NotImplementedError("write your pallas kernel here")


# speedup vs baseline: 5.1672x; 5.1672x over previous
import functools

import jax
import jax.numpy as jnp
from jax import lax
from jax.experimental import pallas as pl
from jax.experimental.pallas import tpu as pltpu
from jax.experimental.pallas import tpu_sc as plsc

N = 10000
D = 128
EPS = 1e-5
NC = 2
NS = 16
NW = NC * NS
CH = 128
NPAD = 10240
RPS = NPAD // NS
SPLIT = 5000
HALF = 5120
HRPS = HALF // NS
JUNK = HALF - 16
NB = 10
BR = N // NB
HB = SPLIT // BR


def _sc_mesh():
  return plsc.VectorSubcoreMesh(core_axis_name="c", subcore_axis_name="s")


def _sc_degree(dst_w, cpt):
  hh = NPAD // 2

  @functools.partial(
      pl.kernel,
      out_type=jax.ShapeDtypeStruct((NW, 2, hh), jnp.float32),
      mesh=_sc_mesh(),
      compiler_params=pltpu.CompilerParams(needs_layout_passes=False),
      scratch_types=[
          pltpu.VMEM((cpt * CH,), jnp.int32),
          pltpu.VMEM((16 * hh,), jnp.float32),
      ],
  )
  def deg_kernel(dst_hbm, out_hbm, dstv, hist):
    c = lax.axis_index("c")
    s = lax.axis_index("s")
    wid = c * NS + s
    pltpu.sync_copy(dst_hbm.at[wid], dstv)
    lanebase = lax.iota(jnp.int32, 16) * hh
    ones = jnp.ones((16,), jnp.float32)

    for h in range(2):
      @pl.loop(0, 16 * hh // 16)
      def _(i):
        off = pl.multiple_of(i * 16, 16)
        hist[pl.ds(off, 16)] = jnp.zeros((16,), jnp.float32)

      @pl.loop(0, cpt * CH // 16)
      def _(i):
        off = pl.multiple_of(i * 16, 16)
        local = dstv[pl.ds(off, 16)] - h * hh
        m = (local >= 0) & (local < hh)
        idx = lanebase + jnp.where(m, local, 0)
        plsc.addupdate_scatter(hist, [idx], ones, mask=m)

      @pl.loop(0, hh // 16)
      def _(i):
        off = pl.multiple_of(i * 16, 16)
        acc = hist[pl.ds(off, 16)]
        for l in range(1, 16):
          acc = acc + hist[pl.ds(l * hh + off, 16)]
        hist[pl.ds(off, 16)] = acc

      pltpu.sync_copy(hist.at[pl.ds(0, hh)], out_hbm.at[wid, h])

  return deg_kernel(dst_w)


def _sc_aggregate(ht, src_w, dst_w, cpt):
  @functools.partial(
      pl.kernel,
      out_type=jax.ShapeDtypeStruct((2, NC, HALF, D), jnp.float32),
      mesh=_sc_mesh(),
      scratch_types=[
          pltpu.VMEM((cpt * CH,), jnp.int32),
          pltpu.VMEM((cpt * CH,), jnp.int32),
          pltpu.VMEM((cpt * CH,), jnp.int32),
          pltpu.VMEM((2, CH, D), jnp.float32),
          pltpu.VMEM((HRPS // 2, D), jnp.float32),
          pltpu.VMEM_SHARED((HALF, D), jnp.float32),
          pltpu.SemaphoreType.DMA,
          pltpu.SemaphoreType.DMA,
      ],
  )
  def agg_kernel(ht_hbm, src_hbm, dst_hbm, out_hbm,
                 srcv, dstv, dstp, rows, zbuf, acc, sem0, sem1):
    c = lax.axis_index("c")
    s = lax.axis_index("s")
    wid = c * NS + s
    base = s * HRPS
    sems = (sem0, sem1)

    @pl.loop(0, HRPS // 2)
    def _(i):
      for k in range(D // 16):
        zbuf[i, pl.ds(16 * k, 16)] = jnp.zeros((16,), jnp.float32)

    pltpu.sync_copy(src_hbm.at[wid], srcv)
    pltpu.sync_copy(dst_hbm.at[wid], dstv)

    for h in range(2):
      @pl.loop(0, cpt * CH // 16)
      def _(i):
        off = pl.multiple_of(i * 16, 16)
        dv = dstv[pl.ds(off, 16)]
        junk = JUNK + (dv & 15)
        if h == 0:
          dstp[pl.ds(off, 16)] = jnp.where(dv < SPLIT, dv, junk)
        else:
          dstp[pl.ds(off, 16)] = jnp.where(dv >= SPLIT, dv - SPLIT, junk)

      pltpu.sync_copy(zbuf, acc.at[pl.ds(base, HRPS // 2)])
      pltpu.sync_copy(zbuf, acc.at[pl.ds(base + HRPS // 2, HRPS // 2)])
      plsc.subcore_barrier()

      def g_desc(j, b):
        off = pl.multiple_of(j * CH, CH)
        return pltpu.make_async_copy(
            ht_hbm.at[srcv.at[pl.ds(off, CH)]], rows.at[b], sems[b])

      def scat(j, b):
        off = pl.multiple_of(j * CH, CH)
        pltpu.sync_copy(rows.at[b], acc.at[dstp.at[pl.ds(off, CH)]], add=True)

      g_desc(0, 0).start()
      g_desc(1, 1).start()

      @pl.loop(0, cpt // 2 - 1)
      def _(g):
        j0 = 2 * g
        g_desc(j0, 0).wait()
        scat(j0, 0)
        g_desc(j0 + 2, 0).start()
        g_desc(j0 + 1, 1).wait()
        scat(j0 + 1, 1)
        g_desc(j0 + 3, 1).start()

      g_desc(cpt - 2, 0).wait()
      scat(cpt - 2, 0)
      g_desc(cpt - 1, 1).wait()
      scat(cpt - 1, 1)

      plsc.subcore_barrier()
      pltpu.sync_copy(acc.at[pl.ds(base, HRPS)],
                      out_hbm.at[h, c, pl.ds(base, HRPS)])

  return agg_kernel(ht, src_w, dst_w)


def _tc_mm_scale(x, W, degp):
  br = 1024

  def body(deg_ref, x_ref, w_ref, ht_ref, dinv_ref):
    d = jnp.sum(deg_ref[...], axis=0) + 1.0
    dinv = lax.rsqrt(d).reshape(br, 1)
    ht_ref[...] = jnp.dot(x_ref[...], w_ref[...],
                          preferred_element_type=jnp.float32) * dinv
    dinv_ref[...] = dinv

  return pl.pallas_call(
      body,
      grid=(NPAD // br,),
      in_specs=[
          pl.BlockSpec((NW, br), lambda i: (0, i)),
          pl.BlockSpec((br, D), lambda i: (i, 0)),
          pl.BlockSpec((D, D), lambda i: (0, 0)),
      ],
      out_specs=[
          pl.BlockSpec((br, D), lambda i: (i, 0)),
          pl.BlockSpec((br, 1), lambda i: (i, 0)),
      ],
      out_shape=[
          jax.ShapeDtypeStruct((N, D), jnp.float32),
          jax.ShapeDtypeStruct((N, 1), jnp.float32),
      ],
  )(degp, x, W)


def _tc_combine(sp, ht, dinv):
  def body(sp_ref, ht_ref, dinv_ref, z_ref, st_ref):
    z = (sp_ref[0, 0] + sp_ref[0, 1] + ht_ref[...]) * dinv_ref[...]
    z_ref[...] = z

    @pl.when(pl.program_id(0) == 0)
    def _():
      st_ref[...] = jnp.zeros_like(st_ref)

    st_ref[...] += jnp.concatenate(
        [jnp.sum(z, axis=0, keepdims=True),
         jnp.sum(z * z, axis=0, keepdims=True)], axis=0)

  return pl.pallas_call(
      body,
      grid=(NB,),
      in_specs=[
          pl.BlockSpec((1, NC, BR, D), lambda i: (i // HB, 0, i % HB, 0)),
          pl.BlockSpec((BR, D), lambda i: (i, 0)),
          pl.BlockSpec((BR, 1), lambda i: (i, 0)),
      ],
      out_specs=[
          pl.BlockSpec((BR, D), lambda i: (i, 0)),
          pl.BlockSpec((2, D), lambda i: (0, 0)),
      ],
      out_shape=[
          jax.ShapeDtypeStruct((N, D), jnp.float32),
          jax.ShapeDtypeStruct((2, D), jnp.float32),
      ],
  )(sp, ht, dinv)


def _tc_bn_relu_mm(z, st, g, be, W, dinv):
  def body(z_ref, st_ref, g_ref, be_ref, w_ref, dinv_ref, o_ref):
    mean = st_ref[0:1, :] * (1.0 / N)
    ex2 = st_ref[1:2, :] * (1.0 / N)
    scale = lax.rsqrt(ex2 - mean * mean + EPS) * g_ref[...]
    r = jnp.maximum((z_ref[...] - mean) * scale + be_ref[...], 0.0)
    o_ref[...] = jnp.dot(r, w_ref[...],
                         preferred_element_type=jnp.float32) * dinv_ref[...]

  return pl.pallas_call(
      body,
      grid=(NB,),
      in_specs=[
          pl.BlockSpec((BR, D), lambda i: (i, 0)),
          pl.BlockSpec((2, D), lambda i: (0, 0)),
          pl.BlockSpec((1, D), lambda i: (0, 0)),
          pl.BlockSpec((1, D), lambda i: (0, 0)),
          pl.BlockSpec((D, D), lambda i: (0, 0)),
          pl.BlockSpec((BR, 1), lambda i: (i, 0)),
      ],
      out_specs=pl.BlockSpec((BR, D), lambda i: (i, 0)),
      out_shape=jax.ShapeDtypeStruct((N, D), jnp.float32),
  )(z, st, g, be, W, dinv)


def _tc_bn_relu(z, st, g, be):
  def body(z_ref, st_ref, g_ref, be_ref, o_ref):
    mean = st_ref[0:1, :] * (1.0 / N)
    ex2 = st_ref[1:2, :] * (1.0 / N)
    scale = lax.rsqrt(ex2 - mean * mean + EPS) * g_ref[...]
    o_ref[...] = jnp.maximum((z_ref[...] - mean) * scale + be_ref[...], 0.0)

  return pl.pallas_call(
      body,
      grid=(NB,),
      in_specs=[
          pl.BlockSpec((BR, D), lambda i: (i, 0)),
          pl.BlockSpec((2, D), lambda i: (0, 0)),
          pl.BlockSpec((1, D), lambda i: (0, 0)),
          pl.BlockSpec((1, D), lambda i: (0, 0)),
      ],
      out_specs=pl.BlockSpec((BR, D), lambda i: (i, 0)),
      out_shape=jax.ShapeDtypeStruct((N, D), jnp.float32),
  )(z, st, g, be)


def kernel(x, edge_index, W0, b0, g0, be0, W1, b1, g1, be1):
  del b0, b1
  E = edge_index.shape[1]
  cpt = -(-E // (NW * CH))
  cpt += cpt % 2
  pad = NW * cpt * CH - E

  src = jnp.concatenate([edge_index[0], jnp.zeros((pad,), jnp.int32)])
  dst = jnp.concatenate(
      [edge_index[1], N + (jnp.arange(pad, dtype=jnp.int32) % 16)])
  src_w = src.reshape(NW, cpt * CH)
  dst_w = dst.reshape(NW, cpt * CH)
  g0r, be0r = g0.reshape(1, D), be0.reshape(1, D)
  g1r, be1r = g1.reshape(1, D), be1.reshape(1, D)

  degp = _sc_degree(dst_w, cpt).reshape(NW, NPAD)
  ht0, dinv = _tc_mm_scale(x, W0, degp)
  sp0 = _sc_aggregate(ht0, src_w, dst_w, cpt)
  z0, st0 = _tc_combine(sp0, ht0, dinv)
  ht1 = _tc_bn_relu_mm(z0, st0, g0r, be0r, W1, dinv)
  sp1 = _sc_aggregate(ht1, src_w, dst_w, cpt)
  z1, st1 = _tc_combine(sp1, ht1, dinv)
  return _tc_bn_relu(z1, st1, g1r, be1r)


# speedup vs baseline: 6.3148x; 1.2221x over previous
import functools

import jax
import jax.numpy as jnp
from jax import lax
from jax.experimental import pallas as pl
from jax.experimental.pallas import tpu as pltpu
from jax.experimental.pallas import tpu_sc as plsc

N = 10000
D = 128
EPS = 1e-5
NC = 2
NS = 16
NW = NC * NS
CH = 128
NPAD = 10240
NPASS = 4
PR = NPAD // NPASS
PRS = PR // NS
NBUF = 2
NB = 10
BR = N // NB
CBR = 1280


def _sc_mesh():
  return plsc.VectorSubcoreMesh(core_axis_name="c", subcore_axis_name="s")


def _sc_degree(dst_w, cpt):
  hh = NPAD // 2

  @functools.partial(
      pl.kernel,
      out_type=jax.ShapeDtypeStruct((NW, 2, hh), jnp.float32),
      mesh=_sc_mesh(),
      compiler_params=pltpu.CompilerParams(needs_layout_passes=False),
      scratch_types=[
          pltpu.VMEM((cpt * CH,), jnp.int32),
          pltpu.VMEM((16 * hh,), jnp.float32),
      ],
  )
  def deg_kernel(dst_hbm, out_hbm, dstv, hist):
    c = lax.axis_index("c")
    s = lax.axis_index("s")
    wid = c * NS + s
    pltpu.sync_copy(dst_hbm.at[wid], dstv)
    lanebase = lax.iota(jnp.int32, 16) * hh
    ones = jnp.ones((16,), jnp.float32)

    for h in range(2):
      @pl.loop(0, 16 * hh // 16)
      def _(i):
        off = pl.multiple_of(i * 16, 16)
        hist[pl.ds(off, 16)] = jnp.zeros((16,), jnp.float32)

      @pl.loop(0, cpt * CH // 16)
      def _(i):
        off = pl.multiple_of(i * 16, 16)
        local = dstv[pl.ds(off, 16)] - h * hh
        m = (local >= 0) & (local < hh)
        idx = lanebase + jnp.where(m, local, 0)
        plsc.addupdate_scatter(hist, [idx], ones, mask=m)

      @pl.loop(0, hh // 16)
      def _(i):
        off = pl.multiple_of(i * 16, 16)
        acc = hist[pl.ds(off, 16)]
        for l in range(1, 16):
          acc = acc + hist[pl.ds(l * hh + off, 16)]
        hist[pl.ds(off, 16)] = acc

      pltpu.sync_copy(hist.at[pl.ds(0, hh)], out_hbm.at[wid, h])

  return deg_kernel(dst_w)


def _sc_aggregate(ht, src_w, dst_w, cpt):
  @functools.partial(
      pl.kernel,
      out_type=jax.ShapeDtypeStruct((NPASS, NC, PR, D), jnp.float32),
      mesh=_sc_mesh(),
      scratch_types=[
          pltpu.VMEM((cpt * CH,), jnp.int32),
          pltpu.VMEM((cpt * CH,), jnp.int32),
          pltpu.VMEM((cpt * CH,), jnp.int32),
          pltpu.VMEM((cpt * CH,), jnp.int32),
          pltpu.VMEM((NBUF, CH, D), jnp.float32),
          pltpu.VMEM((PRS // 2, D), jnp.float32),
          pltpu.VMEM_SHARED((PR, D), jnp.float32),
          pltpu.SemaphoreType.DMA,
          pltpu.SemaphoreType.DMA,
          pltpu.SemaphoreType.DMA,
          pltpu.SemaphoreType.DMA,
      ],
  )
  def agg_kernel(ht_hbm, src_hbm, dst_hbm, out_hbm,
                 srcv, dstv, srcp, dstp, rows, zbuf, acc,
                 gs0, gs1, ss0, ss1):
    gsems = (gs0, gs1)
    ssems = (ss0, ss1)
    c = lax.axis_index("c")
    s = lax.axis_index("s")
    wid = c * NS + s
    base = s * PRS

    @pl.loop(0, PRS // 2)
    def _(i):
      for k in range(D // 16):
        zbuf[i, pl.ds(16 * k, 16)] = jnp.zeros((16,), jnp.float32)

    pltpu.sync_copy(src_hbm.at[wid], srcv)
    pltpu.sync_copy(dst_hbm.at[wid], dstv)

    for p in range(NPASS):
      lo = p * PR

      @pl.loop(0, cpt * CH // 16)
      def _(i):
        off = pl.multiple_of(i * 16, 16)
        dl = dstv[pl.ds(off, 16)] - lo
        sv = srcv[pl.ds(off, 16)]
        m = (dl >= 0) & (dl < PR)
        neg = jnp.full((16,), -1, jnp.int32)
        srcp[pl.ds(off, 16)] = jnp.where(m, sv, neg)
        dstp[pl.ds(off, 16)] = jnp.where(m, dl, neg)

      pltpu.sync_copy(zbuf, acc.at[pl.ds(base, PRS // 2)])
      pltpu.sync_copy(zbuf, acc.at[pl.ds(base + PRS // 2, PRS // 2)])
      plsc.subcore_barrier()

      def gcopy(j, b):
        off = pl.multiple_of(j * CH, CH)
        return pltpu.make_async_copy(
            ht_hbm.at[plsc.Indices(srcp.at[pl.ds(off, CH)],
                                   ignored_value=-1)],
            rows.at[b], gsems[b])

      def scopy(j, b):
        off = pl.multiple_of(j * CH, CH)
        return pltpu.make_async_copy(
            rows.at[b],
            acc.at[plsc.Indices(dstp.at[pl.ds(off, CH)], ignored_value=-1)],
            ssems[b])

      for b in range(NBUF):
        gcopy(b, b).start()

      @pl.loop(0, cpt // NBUF - 1)
      def _(g):
        for b in range(NBUF):
          j = NBUF * g + b
          gcopy(j, b).wait()
          scopy(j, b).start(add=True)
        for b in range(NBUF):
          j = NBUF * g + b
          scopy(j, b).wait()
          gcopy(j + NBUF, b).start()

      for b in range(NBUF):
        j = cpt - NBUF + b
        gcopy(j, b).wait()
        scopy(j, b).start(add=True)
      for b in range(NBUF):
        scopy(cpt - NBUF + b, b).wait()

      plsc.subcore_barrier()
      pltpu.sync_copy(acc.at[pl.ds(base, PRS)],
                      out_hbm.at[p, c, pl.ds(base, PRS)])

  return agg_kernel(ht, src_w, dst_w)


def _tc_mm_scale(x, W, degp):
  br = 1024

  def body(deg_ref, x_ref, w_ref, ht_ref, dinv_ref):
    d = jnp.sum(deg_ref[...], axis=0) + 1.0
    dinv = lax.rsqrt(d).reshape(br, 1)
    ht_ref[...] = jnp.dot(x_ref[...], w_ref[...],
                          preferred_element_type=jnp.float32) * dinv
    dinv_ref[...] = dinv

  return pl.pallas_call(
      body,
      grid=(NPAD // br,),
      in_specs=[
          pl.BlockSpec((NW, br), lambda i: (0, i)),
          pl.BlockSpec((br, D), lambda i: (i, 0)),
          pl.BlockSpec((D, D), lambda i: (0, 0)),
      ],
      out_specs=[
          pl.BlockSpec((br, D), lambda i: (i, 0)),
          pl.BlockSpec((br, 1), lambda i: (i, 0)),
      ],
      out_shape=[
          jax.ShapeDtypeStruct((N, D), jnp.float32),
          jax.ShapeDtypeStruct((N, 1), jnp.float32),
      ],
  )(degp, x, W)


def _tc_combine(sp, ht, dinv):
  bpp = PR // CBR

  def body(sp_ref, ht_ref, dinv_ref, z_ref, st_ref):
    i = pl.program_id(0)
    z = (sp_ref[0, 0] + sp_ref[0, 1] + ht_ref[...]) * dinv_ref[...]
    z_ref[...] = z

    @pl.when(i == 0)
    def _():
      st_ref[...] = jnp.zeros_like(st_ref)

    rowid = lax.broadcasted_iota(jnp.int32, (CBR, 1), 0) + i * CBR
    zm = jnp.where(rowid < N, z, 0.0)
    st_ref[...] += jnp.concatenate(
        [jnp.sum(zm, axis=0, keepdims=True),
         jnp.sum(zm * zm, axis=0, keepdims=True)], axis=0)

  return pl.pallas_call(
      body,
      grid=(NPAD // CBR,),
      in_specs=[
          pl.BlockSpec((1, NC, CBR, D), lambda i: (i // bpp, 0, i % bpp, 0)),
          pl.BlockSpec((CBR, D), lambda i: (i, 0)),
          pl.BlockSpec((CBR, 1), lambda i: (i, 0)),
      ],
      out_specs=[
          pl.BlockSpec((CBR, D), lambda i: (i, 0)),
          pl.BlockSpec((2, D), lambda i: (0, 0)),
      ],
      out_shape=[
          jax.ShapeDtypeStruct((N, D), jnp.float32),
          jax.ShapeDtypeStruct((2, D), jnp.float32),
      ],
  )(sp, ht, dinv)


def _tc_bn_relu_mm(z, st, g, be, W, dinv):
  def body(z_ref, st_ref, g_ref, be_ref, w_ref, dinv_ref, o_ref):
    mean = st_ref[0:1, :] * (1.0 / N)
    ex2 = st_ref[1:2, :] * (1.0 / N)
    scale = lax.rsqrt(ex2 - mean * mean + EPS) * g_ref[...]
    r = jnp.maximum((z_ref[...] - mean) * scale + be_ref[...], 0.0)
    o_ref[...] = jnp.dot(r, w_ref[...],
                         preferred_element_type=jnp.float32) * dinv_ref[...]

  return pl.pallas_call(
      body,
      grid=(NB,),
      in_specs=[
          pl.BlockSpec((BR, D), lambda i: (i, 0)),
          pl.BlockSpec((2, D), lambda i: (0, 0)),
          pl.BlockSpec((1, D), lambda i: (0, 0)),
          pl.BlockSpec((1, D), lambda i: (0, 0)),
          pl.BlockSpec((D, D), lambda i: (0, 0)),
          pl.BlockSpec((BR, 1), lambda i: (i, 0)),
      ],
      out_specs=pl.BlockSpec((BR, D), lambda i: (i, 0)),
      out_shape=jax.ShapeDtypeStruct((N, D), jnp.float32),
  )(z, st, g, be, W, dinv)


def _tc_bn_relu(z, st, g, be):
  def body(z_ref, st_ref, g_ref, be_ref, o_ref):
    mean = st_ref[0:1, :] * (1.0 / N)
    ex2 = st_ref[1:2, :] * (1.0 / N)
    scale = lax.rsqrt(ex2 - mean * mean + EPS) * g_ref[...]
    o_ref[...] = jnp.maximum((z_ref[...] - mean) * scale + be_ref[...], 0.0)

  return pl.pallas_call(
      body,
      grid=(NB,),
      in_specs=[
          pl.BlockSpec((BR, D), lambda i: (i, 0)),
          pl.BlockSpec((2, D), lambda i: (0, 0)),
          pl.BlockSpec((1, D), lambda i: (0, 0)),
          pl.BlockSpec((1, D), lambda i: (0, 0)),
      ],
      out_specs=pl.BlockSpec((BR, D), lambda i: (i, 0)),
      out_shape=jax.ShapeDtypeStruct((N, D), jnp.float32),
  )(z, st, g, be)


def kernel(x, edge_index, W0, b0, g0, be0, W1, b1, g1, be1):
  del b0, b1
  E = edge_index.shape[1]
  cpt = -(-E // (NW * CH))
  cpt += cpt % 2
  pad = NW * cpt * CH - E

  src = jnp.concatenate([edge_index[0], jnp.zeros((pad,), jnp.int32)])
  dst = jnp.concatenate(
      [edge_index[1], N + (jnp.arange(pad, dtype=jnp.int32) % 16)])
  src_w = src.reshape(NW, cpt * CH)
  dst_w = dst.reshape(NW, cpt * CH)
  g0r, be0r = g0.reshape(1, D), be0.reshape(1, D)
  g1r, be1r = g1.reshape(1, D), be1.reshape(1, D)

  degp = _sc_degree(dst_w, cpt).reshape(NW, NPAD)
  ht0, dinv = _tc_mm_scale(x, W0, degp)
  sp0 = _sc_aggregate(ht0, src_w, dst_w, cpt)
  z0, st0 = _tc_combine(sp0, ht0, dinv)
  ht1 = _tc_bn_relu_mm(z0, st0, g0r, be0r, W1, dinv)
  sp1 = _sc_aggregate(ht1, src_w, dst_w, cpt)
  z1, st1 = _tc_combine(sp1, ht1, dinv)
  return _tc_bn_relu(z1, st1, g1r, be1r)


# speedup vs baseline: 7.9867x; 1.2648x over previous
import functools

import jax
import jax.numpy as jnp
from jax import lax
from jax.experimental import pallas as pl
from jax.experimental.pallas import tpu as pltpu
from jax.experimental.pallas import tpu_sc as plsc

N = 10000
D = 128
EPS = 1e-5
NC = 2
NS = 16
NW = NC * NS
CH = 128
NPAD = 10240
NPASS = 2
PR = 5120
PRS = PR // NS
NBUF = 2
NB = 10
BR = N // NB
CBR = 1280


def _sc_mesh():
  return plsc.VectorSubcoreMesh(core_axis_name="c", subcore_axis_name="s")


def _sc_degree(dst_w, cpt):
  hh = NPAD // 2

  @functools.partial(
      pl.kernel,
      out_type=jax.ShapeDtypeStruct((NW, 2, hh), jnp.float32),
      mesh=_sc_mesh(),
      compiler_params=pltpu.CompilerParams(needs_layout_passes=False),
      scratch_types=[
          pltpu.VMEM((cpt * CH,), jnp.int32),
          pltpu.VMEM((16 * hh,), jnp.float32),
      ],
  )
  def deg_kernel(dst_hbm, out_hbm, dstv, hist):
    c = lax.axis_index("c")
    s = lax.axis_index("s")
    wid = c * NS + s

    @pl.loop(0, cpt * CH // 2048)
    def _(i):
      off = pl.multiple_of(i * 2048, 2048)
      pltpu.sync_copy(dst_hbm.at[wid, pl.ds(off, 2048)],
                      dstv.at[pl.ds(off, 2048)])

    lanebase = lax.iota(jnp.int32, 16) * hh
    ones = jnp.ones((16,), jnp.float32)

    for h in range(2):
      @pl.loop(0, 16 * hh // 16)
      def _(i):
        off = pl.multiple_of(i * 16, 16)
        hist[pl.ds(off, 16)] = jnp.zeros((16,), jnp.float32)

      @pl.loop(0, cpt * CH // 16)
      def _(i):
        off = pl.multiple_of(i * 16, 16)
        local = dstv[pl.ds(off, 16)] - h * hh
        m = (local >= 0) & (local < hh)
        idx = lanebase + jnp.where(m, local, 0)
        plsc.addupdate_scatter(hist, [idx], ones, mask=m)

      @pl.loop(0, hh // 16)
      def _(i):
        off = pl.multiple_of(i * 16, 16)
        acc = hist[pl.ds(off, 16)]
        for l in range(1, 16):
          acc = acc + hist[pl.ds(l * hh + off, 16)]
        hist[pl.ds(off, 16)] = acc

      @pl.loop(0, hh // 1024)
      def _(k):
        off = pl.multiple_of(k * 1024, 1024)
        pltpu.sync_copy(hist.at[pl.ds(off, 1024)],
                        out_hbm.at[wid, h, pl.ds(off, 1024)])

  return deg_kernel(dst_w)


def _sc_aggregate(ht, src_w, dst_w, cpt):
  @functools.partial(
      pl.kernel,
      out_type=jax.ShapeDtypeStruct((NPASS, NC, PR, D), jnp.float32),
      mesh=_sc_mesh(),
      scratch_types=[
          pltpu.VMEM((cpt * CH,), jnp.int32),
          pltpu.VMEM((cpt * CH,), jnp.int32),
          pltpu.VMEM((cpt * CH,), jnp.int32),
          pltpu.VMEM((cpt * CH,), jnp.int32),
          pltpu.VMEM((NBUF, CH, D), jnp.float32),
          pltpu.VMEM((32, D), jnp.float32),
          pltpu.VMEM_SHARED((PR, D), jnp.float32),
          pltpu.SemaphoreType.DMA,
          pltpu.SemaphoreType.DMA,
          pltpu.SemaphoreType.DMA,
          pltpu.SemaphoreType.DMA,
      ],
  )
  def agg_kernel(ht_hbm, src_hbm, dst_hbm, out_hbm,
                 srcv, dstv, srcp, dstp, rows, zbuf, acc,
                 gs0, gs1, ss0, ss1):
    gsems = (gs0, gs1)
    ssems = (ss0, ss1)
    c = lax.axis_index("c")
    s = lax.axis_index("s")
    wid = c * NS + s
    base = s * PRS

    @pl.loop(0, 32)
    def _(i):
      for k in range(D // 16):
        zbuf[i, pl.ds(16 * k, 16)] = jnp.zeros((16,), jnp.float32)

    @pl.loop(0, cpt * CH // 2048)
    def _(i):
      off = pl.multiple_of(i * 2048, 2048)
      pltpu.sync_copy(src_hbm.at[wid, pl.ds(off, 2048)],
                      srcv.at[pl.ds(off, 2048)])
      pltpu.sync_copy(dst_hbm.at[wid, pl.ds(off, 2048)],
                      dstv.at[pl.ds(off, 2048)])

    @pl.loop(0, NPASS)
    def _(p):
      lo = p * PR

      @pl.loop(0, cpt * CH // 16)
      def _(i):
        off = pl.multiple_of(i * 16, 16)
        dl = dstv[pl.ds(off, 16)] - lo
        sv = srcv[pl.ds(off, 16)]
        m = (dl >= 0) & (dl < PR)
        neg = jnp.full((16,), -1, jnp.int32)
        srcp[pl.ds(off, 16)] = jnp.where(m, sv, neg)
        dstp[pl.ds(off, 16)] = jnp.where(m, dl, neg)

      @pl.loop(0, PRS // 32)
      def _(i):
        pltpu.sync_copy(zbuf, acc.at[pl.ds(base + i * 32, 32)])

      plsc.subcore_barrier()

      def gcopy(j, b):
        off = pl.multiple_of(j * CH, CH)
        return pltpu.make_async_copy(
            ht_hbm.at[plsc.Indices(srcp.at[pl.ds(off, CH)],
                                   ignored_value=-1)],
            rows.at[b], gsems[b])

      def scopy(j, b):
        off = pl.multiple_of(j * CH, CH)
        return pltpu.make_async_copy(
            rows.at[b],
            acc.at[plsc.Indices(dstp.at[pl.ds(off, CH)], ignored_value=-1)],
            ssems[b])

      for b in range(NBUF):
        gcopy(b, b).start()

      @pl.loop(0, cpt // NBUF - 1)
      def _(g):
        for b in range(NBUF):
          j = NBUF * g + b
          gcopy(j, b).wait()
          scopy(j, b).start(add=True)
        for b in range(NBUF):
          j = NBUF * g + b
          scopy(j, b).wait()
          gcopy(j + NBUF, b).start()

      for b in range(NBUF):
        j = cpt - NBUF + b
        gcopy(j, b).wait()
        scopy(j, b).start(add=True)
      for b in range(NBUF):
        scopy(cpt - NBUF + b, b).wait()

      plsc.subcore_barrier()

      @pl.loop(0, PRS // 32)
      def _(i):
        pltpu.sync_copy(acc.at[pl.ds(base + i * 32, 32)],
                        out_hbm.at[p, c, pl.ds(base + i * 32, 32)])

  return agg_kernel(ht, src_w, dst_w)


def _tc_mm_scale(x, W, degp):
  br = 1024

  def body(deg_ref, x_ref, w_ref, ht_ref, dinv_ref):
    d = jnp.sum(deg_ref[...], axis=0) + 1.0
    dinv = lax.rsqrt(d).reshape(br, 1)
    ht_ref[...] = jnp.dot(x_ref[...], w_ref[...],
                          preferred_element_type=jnp.float32) * dinv
    dinv_ref[...] = dinv

  return pl.pallas_call(
      body,
      grid=(NPAD // br,),
      in_specs=[
          pl.BlockSpec((NW, br), lambda i: (0, i)),
          pl.BlockSpec((br, D), lambda i: (i, 0)),
          pl.BlockSpec((D, D), lambda i: (0, 0)),
      ],
      out_specs=[
          pl.BlockSpec((br, D), lambda i: (i, 0)),
          pl.BlockSpec((br, 1), lambda i: (i, 0)),
      ],
      out_shape=[
          jax.ShapeDtypeStruct((N, D), jnp.float32),
          jax.ShapeDtypeStruct((N, 1), jnp.float32),
      ],
  )(degp, x, W)


def _tc_combine(sp, ht, dinv):
  bpp = PR // CBR

  def body(sp_ref, ht_ref, dinv_ref, z_ref, st_ref):
    i = pl.program_id(0)
    z = (sp_ref[0, 0] + sp_ref[0, 1] + ht_ref[...]) * dinv_ref[...]
    z_ref[...] = z

    @pl.when(i == 0)
    def _():
      st_ref[...] = jnp.zeros_like(st_ref)

    rowid = lax.broadcasted_iota(jnp.int32, (CBR, 1), 0) + i * CBR
    zm = jnp.where(rowid < N, z, 0.0)
    st_ref[...] += jnp.concatenate(
        [jnp.sum(zm, axis=0, keepdims=True),
         jnp.sum(zm * zm, axis=0, keepdims=True)], axis=0)

  return pl.pallas_call(
      body,
      grid=(NPAD // CBR,),
      in_specs=[
          pl.BlockSpec((1, NC, CBR, D), lambda i: (i // bpp, 0, i % bpp, 0)),
          pl.BlockSpec((CBR, D), lambda i: (i, 0)),
          pl.BlockSpec((CBR, 1), lambda i: (i, 0)),
      ],
      out_specs=[
          pl.BlockSpec((CBR, D), lambda i: (i, 0)),
          pl.BlockSpec((2, D), lambda i: (0, 0)),
      ],
      out_shape=[
          jax.ShapeDtypeStruct((N, D), jnp.float32),
          jax.ShapeDtypeStruct((2, D), jnp.float32),
      ],
  )(sp, ht, dinv)


def _tc_bn_relu_mm(z, st, g, be, W, dinv):
  def body(z_ref, st_ref, g_ref, be_ref, w_ref, dinv_ref, o_ref):
    mean = st_ref[0:1, :] * (1.0 / N)
    ex2 = st_ref[1:2, :] * (1.0 / N)
    scale = lax.rsqrt(ex2 - mean * mean + EPS) * g_ref[...]
    r = jnp.maximum((z_ref[...] - mean) * scale + be_ref[...], 0.0)
    o_ref[...] = jnp.dot(r, w_ref[...],
                         preferred_element_type=jnp.float32) * dinv_ref[...]

  return pl.pallas_call(
      body,
      grid=(NB,),
      in_specs=[
          pl.BlockSpec((BR, D), lambda i: (i, 0)),
          pl.BlockSpec((2, D), lambda i: (0, 0)),
          pl.BlockSpec((1, D), lambda i: (0, 0)),
          pl.BlockSpec((1, D), lambda i: (0, 0)),
          pl.BlockSpec((D, D), lambda i: (0, 0)),
          pl.BlockSpec((BR, 1), lambda i: (i, 0)),
      ],
      out_specs=pl.BlockSpec((BR, D), lambda i: (i, 0)),
      out_shape=jax.ShapeDtypeStruct((N, D), jnp.float32),
  )(z, st, g, be, W, dinv)


def _tc_bn_relu(z, st, g, be):
  def body(z_ref, st_ref, g_ref, be_ref, o_ref):
    mean = st_ref[0:1, :] * (1.0 / N)
    ex2 = st_ref[1:2, :] * (1.0 / N)
    scale = lax.rsqrt(ex2 - mean * mean + EPS) * g_ref[...]
    o_ref[...] = jnp.maximum((z_ref[...] - mean) * scale + be_ref[...], 0.0)

  return pl.pallas_call(
      body,
      grid=(NB,),
      in_specs=[
          pl.BlockSpec((BR, D), lambda i: (i, 0)),
          pl.BlockSpec((2, D), lambda i: (0, 0)),
          pl.BlockSpec((1, D), lambda i: (0, 0)),
          pl.BlockSpec((1, D), lambda i: (0, 0)),
      ],
      out_specs=pl.BlockSpec((BR, D), lambda i: (i, 0)),
      out_shape=jax.ShapeDtypeStruct((N, D), jnp.float32),
  )(z, st, g, be)


def kernel(x, edge_index, W0, b0, g0, be0, W1, b1, g1, be1):
  del b0, b1
  E = edge_index.shape[1]
  cpt = -(-E // (NW * CH))
  cpt += cpt % 2
  pad = NW * cpt * CH - E

  src = jnp.concatenate([edge_index[0], jnp.zeros((pad,), jnp.int32)])
  dst = jnp.concatenate(
      [edge_index[1], N + (jnp.arange(pad, dtype=jnp.int32) % 16)])
  src_w = src.reshape(NW, cpt * CH)
  dst_w = dst.reshape(NW, cpt * CH)
  g0r, be0r = g0.reshape(1, D), be0.reshape(1, D)
  g1r, be1r = g1.reshape(1, D), be1.reshape(1, D)

  degp = _sc_degree(dst_w, cpt).reshape(NW, NPAD)
  ht0, dinv = _tc_mm_scale(x, W0, degp)
  sp0 = _sc_aggregate(ht0, src_w, dst_w, cpt)
  z0, st0 = _tc_combine(sp0, ht0, dinv)
  ht1 = _tc_bn_relu_mm(z0, st0, g0r, be0r, W1, dinv)
  sp1 = _sc_aggregate(ht1, src_w, dst_w, cpt)
  z1, st1 = _tc_combine(sp1, ht1, dinv)
  return _tc_bn_relu(z1, st1, g1r, be1r)


# speedup vs baseline: 12.3937x; 1.5518x over previous
import functools

import jax
import jax.numpy as jnp
from jax import lax
from jax.experimental import pallas as pl
from jax.experimental.pallas import tpu as pltpu
from jax.experimental.pallas import tpu_sc as plsc

N = 10000
D = 128
EPS = 1e-5
NC = 2
NS = 16
NW = NC * NS
CH = 128
NPAD = 10240
NPASS = 2
PR = 5120
PRS = PR // NS
NBUF = 2
NB = 10
BR = N // NB
CBR = 1280


def _sc_mesh():
  return plsc.VectorSubcoreMesh(core_axis_name="c", subcore_axis_name="s")


def _sc_degree(dst_w, cpt):
  hh = NPAD // 2

  @functools.partial(
      pl.kernel,
      out_type=jax.ShapeDtypeStruct((NW, 2, hh), jnp.float32),
      mesh=_sc_mesh(),
      compiler_params=pltpu.CompilerParams(needs_layout_passes=False),
      scratch_types=[
          pltpu.VMEM((cpt * CH,), jnp.int32),
          pltpu.VMEM((16 * hh,), jnp.float32),
      ],
  )
  def deg_kernel(dst_hbm, out_hbm, dstv, hist):
    c = lax.axis_index("c")
    s = lax.axis_index("s")
    wid = c * NS + s

    @pl.loop(0, cpt * CH // 2048)
    def _(i):
      off = pl.multiple_of(i * 2048, 2048)
      pltpu.sync_copy(dst_hbm.at[wid, pl.ds(off, 2048)],
                      dstv.at[pl.ds(off, 2048)])

    lanebase = lax.iota(jnp.int32, 16) * hh
    ones = jnp.ones((16,), jnp.float32)

    for h in range(2):
      @pl.loop(0, 16 * hh // 16)
      def _(i):
        off = pl.multiple_of(i * 16, 16)
        hist[pl.ds(off, 16)] = jnp.zeros((16,), jnp.float32)

      @pl.loop(0, cpt * CH // 16)
      def _(i):
        off = pl.multiple_of(i * 16, 16)
        local = dstv[pl.ds(off, 16)] - h * hh
        m = (local >= 0) & (local < hh)
        idx = lanebase + jnp.where(m, local, 0)
        plsc.addupdate_scatter(hist, [idx], ones, mask=m)

      @pl.loop(0, hh // 16)
      def _(i):
        off = pl.multiple_of(i * 16, 16)
        acc = hist[pl.ds(off, 16)]
        for l in range(1, 16):
          acc = acc + hist[pl.ds(l * hh + off, 16)]
        hist[pl.ds(off, 16)] = acc

      @pl.loop(0, hh // 1024)
      def _(k):
        off = pl.multiple_of(k * 1024, 1024)
        pltpu.sync_copy(hist.at[pl.ds(off, 1024)],
                        out_hbm.at[wid, h, pl.ds(off, 1024)])

  return deg_kernel(dst_w)


def _sc_aggregate(ht, src_w, dst_w, cptm):
  @functools.partial(
      pl.kernel,
      out_type=jax.ShapeDtypeStruct((NPASS, NC, PR, D), jnp.float32),
      mesh=_sc_mesh(),
      scratch_types=[
          pltpu.VMEM((cptm * CH,), jnp.int32),
          pltpu.VMEM((cptm * CH,), jnp.int32),
          pltpu.VMEM((cptm * CH,), jnp.int32),
          pltpu.VMEM((cptm * CH,), jnp.int32),
          pltpu.VMEM((NBUF, CH, D), jnp.float32),
          pltpu.VMEM((8, D), jnp.float32),
          pltpu.VMEM_SHARED((PR, D), jnp.float32),
          pltpu.SemaphoreType.DMA,
          pltpu.SemaphoreType.DMA,
          pltpu.SemaphoreType.DMA,
          pltpu.SemaphoreType.DMA,
      ],
  )
  def agg_kernel(ht_hbm, src_hbm, dst_hbm, out_hbm,
                 srcv, dstv, srcp, dstp, rows, zbuf, acc,
                 gs0, gs1, ss0, ss1):
    gsems = (gs0, gs1)
    ssems = (ss0, ss1)
    c = lax.axis_index("c")
    s = lax.axis_index("s")
    wid = c * NS + s
    base = s * PRS
    cptc = cptm

    @pl.loop(0, 8)
    def _(i):
      for k in range(D // 16):
        zbuf[i, pl.ds(16 * k, 16)] = jnp.zeros((16,), jnp.float32)

    @pl.loop(0, cptc * CH // 1024)
    def _(i):
      off = pl.multiple_of(i * 1024, 1024)
      pltpu.sync_copy(src_hbm.at[wid, pl.ds(off, 1024)],
                      srcv.at[pl.ds(off, 1024)])
      pltpu.sync_copy(dst_hbm.at[wid, pl.ds(off, 1024)],
                      dstv.at[pl.ds(off, 1024)])

    @pl.loop(0, NPASS)
    def _(p):
      lo = p * PR

      @pl.loop(0, cptc * CH // 16)
      def _(i):
        off = pl.multiple_of(i * 16, 16)
        dl = dstv[pl.ds(off, 16)] - lo
        sv = srcv[pl.ds(off, 16)]
        m = (dl >= 0) & (dl < PR)
        neg = jnp.full((16,), -1, jnp.int32)
        srcp[pl.ds(off, 16)] = jnp.where(m, sv, neg)
        dstp[pl.ds(off, 16)] = jnp.where(m, dl, neg)

      @pl.loop(0, PRS // 8)
      def _(i):
        pltpu.sync_copy(zbuf, acc.at[pl.ds(base + i * 8, 8)])

      plsc.subcore_barrier()

      def gcopy(j, b):
        off = pl.multiple_of(j * CH, CH)
        return pltpu.make_async_copy(
            ht_hbm.at[plsc.Indices(srcp.at[pl.ds(off, CH)],
                                   ignored_value=-1)],
            rows.at[b], gsems[b])

      def scopy(j, b):
        off = pl.multiple_of(j * CH, CH)
        return pltpu.make_async_copy(
            rows.at[b],
            acc.at[plsc.Indices(dstp.at[pl.ds(off, CH)], ignored_value=-1)],
            ssems[b])

      for b in range(NBUF):
        gcopy(b, b).start()

      @pl.loop(0, cptc // NBUF)
      def _(g):
        for b in range(NBUF):
          j = NBUF * g + b
          gcopy(j, b).wait()
          scopy(j, b).start(add=True)
        for b in range(NBUF):
          j = NBUF * g + b
          scopy(j, b).wait()

          @pl.when(j + NBUF < cptc)
          def _(b=b, j=j):
            gcopy(j + NBUF, b).start()

      plsc.subcore_barrier()

      @pl.loop(0, PRS // 8)
      def _(i):
        pltpu.sync_copy(acc.at[pl.ds(base + i * 8, 8)],
                        out_hbm.at[p, c, pl.ds(base + i * 8, 8)])

  return agg_kernel(ht, src_w, dst_w)


def _tc_mm_scale(x, W, degp):
  br = 1024

  def body(deg_ref, x_ref, w_ref, ht_ref, dinv_ref):
    d = jnp.sum(deg_ref[...], axis=0) + 1.0
    dinv = lax.rsqrt(d).reshape(br, 1)
    ht_ref[...] = jnp.dot(x_ref[...], w_ref[...],
                          preferred_element_type=jnp.float32) * dinv
    dinv_ref[...] = dinv

  return pl.pallas_call(
      body,
      grid=(NPAD // br,),
      in_specs=[
          pl.BlockSpec((NW, br), lambda i: (0, i)),
          pl.BlockSpec((br, D), lambda i: (i, 0)),
          pl.BlockSpec((D, D), lambda i: (0, 0)),
      ],
      out_specs=[
          pl.BlockSpec((br, D), lambda i: (i, 0)),
          pl.BlockSpec((br, 1), lambda i: (i, 0)),
      ],
      out_shape=[
          jax.ShapeDtypeStruct((N, D), jnp.float32),
          jax.ShapeDtypeStruct((N, 1), jnp.float32),
      ],
  )(degp, x, W)


def _tc_combine(sp, ht, dinv):
  bpp = PR // CBR

  def body(sp_ref, ht_ref, dinv_ref, z_ref, st_ref):
    i = pl.program_id(0)
    z = (sp_ref[0, 0] + sp_ref[0, 1] + ht_ref[...]) * dinv_ref[...]
    z_ref[...] = z

    @pl.when(i == 0)
    def _():
      st_ref[...] = jnp.zeros_like(st_ref)

    rowid = lax.broadcasted_iota(jnp.int32, (CBR, 1), 0) + i * CBR
    zm = jnp.where(rowid < N, z, 0.0)
    st_ref[...] += jnp.concatenate(
        [jnp.sum(zm, axis=0, keepdims=True),
         jnp.sum(zm * zm, axis=0, keepdims=True)], axis=0)

  return pl.pallas_call(
      body,
      grid=(NPAD // CBR,),
      in_specs=[
          pl.BlockSpec((1, NC, CBR, D), lambda i: (i // bpp, 0, i % bpp, 0)),
          pl.BlockSpec((CBR, D), lambda i: (i, 0)),
          pl.BlockSpec((CBR, 1), lambda i: (i, 0)),
      ],
      out_specs=[
          pl.BlockSpec((CBR, D), lambda i: (i, 0)),
          pl.BlockSpec((2, D), lambda i: (0, 0)),
      ],
      out_shape=[
          jax.ShapeDtypeStruct((N, D), jnp.float32),
          jax.ShapeDtypeStruct((2, D), jnp.float32),
      ],
  )(sp, ht, dinv)


def _tc_bn_relu_mm(z, st, g, be, W, dinv):
  def body(z_ref, st_ref, g_ref, be_ref, w_ref, dinv_ref, o_ref):
    mean = st_ref[0:1, :] * (1.0 / N)
    ex2 = st_ref[1:2, :] * (1.0 / N)
    scale = lax.rsqrt(ex2 - mean * mean + EPS) * g_ref[...]
    r = jnp.maximum((z_ref[...] - mean) * scale + be_ref[...], 0.0)
    o_ref[...] = jnp.dot(r, w_ref[...],
                         preferred_element_type=jnp.float32) * dinv_ref[...]

  return pl.pallas_call(
      body,
      grid=(NB,),
      in_specs=[
          pl.BlockSpec((BR, D), lambda i: (i, 0)),
          pl.BlockSpec((2, D), lambda i: (0, 0)),
          pl.BlockSpec((1, D), lambda i: (0, 0)),
          pl.BlockSpec((1, D), lambda i: (0, 0)),
          pl.BlockSpec((D, D), lambda i: (0, 0)),
          pl.BlockSpec((BR, 1), lambda i: (i, 0)),
      ],
      out_specs=pl.BlockSpec((BR, D), lambda i: (i, 0)),
      out_shape=jax.ShapeDtypeStruct((N, D), jnp.float32),
  )(z, st, g, be, W, dinv)


def _tc_bn_relu(z, st, g, be):
  def body(z_ref, st_ref, g_ref, be_ref, o_ref):
    mean = st_ref[0:1, :] * (1.0 / N)
    ex2 = st_ref[1:2, :] * (1.0 / N)
    scale = lax.rsqrt(ex2 - mean * mean + EPS) * g_ref[...]
    o_ref[...] = jnp.maximum((z_ref[...] - mean) * scale + be_ref[...], 0.0)

  return pl.pallas_call(
      body,
      grid=(NB,),
      in_specs=[
          pl.BlockSpec((BR, D), lambda i: (i, 0)),
          pl.BlockSpec((2, D), lambda i: (0, 0)),
          pl.BlockSpec((1, D), lambda i: (0, 0)),
          pl.BlockSpec((1, D), lambda i: (0, 0)),
      ],
      out_specs=pl.BlockSpec((BR, D), lambda i: (i, 0)),
      out_shape=jax.ShapeDtypeStruct((N, D), jnp.float32),
  )(z, st, g, be)


def kernel(x, edge_index, W0, b0, g0, be0, W1, b1, g1, be1):
  del b0, b1
  E = edge_index.shape[1]
  cpt = -(-E // (NW * CH))
  cpt += cpt % 2
  pad = NW * cpt * CH - E

  src = jnp.concatenate([edge_index[0], jnp.zeros((pad,), jnp.int32)])
  dst = jnp.concatenate(
      [edge_index[1], N + (jnp.arange(pad, dtype=jnp.int32) % 16)])
  dst_w = dst.reshape(NW, cpt * CH)

  tot = 2 * cpt
  cpt0 = max(2 * NBUF, (int(tot * 0.35) // 8) * 8)
  cpt1 = tot - cpt0
  cap0 = NS * cpt0 * CH
  du = jnp.concatenate(
      [edge_index[1], jnp.full((pad,), 2 * NPAD, jnp.int32)])
  s0 = src[:cap0].reshape(NS, cpt0 * CH)
  d0 = du[:cap0].reshape(NS, cpt0 * CH)
  s1 = src[cap0:].reshape(NS, cpt1 * CH)
  d1 = du[cap0:].reshape(NS, cpt1 * CH)
  colpad = (max(cpt0, cpt1) - cpt0) * CH
  s0 = jnp.pad(s0, ((0, 0), (0, colpad)))
  d0 = jnp.pad(d0, ((0, 0), (0, colpad)), constant_values=2 * NPAD)
  src_u = jnp.concatenate([s0, s1], axis=0)
  dst_u = jnp.concatenate([d0, d1], axis=0)

  g0r, be0r = g0.reshape(1, D), be0.reshape(1, D)
  g1r, be1r = g1.reshape(1, D), be1.reshape(1, D)

  degp = _sc_degree(dst_w, cpt).reshape(NW, NPAD)
  ht0, dinv = _tc_mm_scale(x, W0, degp)
  sp0 = _sc_aggregate(ht0, src_u, dst_u, max(cpt0, cpt1))
  z0, st0 = _tc_combine(sp0, ht0, dinv)
  ht1 = _tc_bn_relu_mm(z0, st0, g0r, be0r, W1, dinv)
  sp1 = _sc_aggregate(ht1, src_u, dst_u, max(cpt0, cpt1))
  z1, st1 = _tc_combine(sp1, ht1, dinv)
  return _tc_bn_relu(z1, st1, g1r, be1r)
